# baseline (speedup 1.0000x reference)
import functools

import jax
import jax.numpy as jnp
from jax import lax
from jax.experimental import pallas as pl
from jax.experimental.pallas import tpu as pltpu
from jax.experimental.pallas import tpu_sc as plsc

B = 16384
D = 64
NC = 2
NS = 16
NW = NC * NS
BPW = B // NW
NCHUNK = 4
CH = BPW // NCHUNK


def _sc_body(src_hbm, tgt_hbm, table_hbm, wb_hbm, out_hbm,
             sidx, tidx, srows, trows, outv, wbv, sem):
    cid = lax.axis_index("c")
    sid = lax.axis_index("s")
    wid = sid * NC + cid
    base = wid * BPW

    pltpu.sync_copy(src_hbm.at[wid], sidx)
    pltpu.sync_copy(tgt_hbm.at[wid], tidx)
    pltpu.sync_copy(wb_hbm, wbv)

    copies = []
    for j in range(NCHUNK):
        copies.append(pltpu.async_copy(
            table_hbm.at[sidx.at[j]], srows.at[pl.ds(j * CH, CH)], sem))
        copies.append(pltpu.async_copy(
            table_hbm.at[tidx.at[j]], trows.at[pl.ds(j * CH, CH)], sem))
    for c in copies:
        c.wait()

    wv = wbv[...]
    w = wv[0]
    bb = wv[1]
    lanes = lax.iota(jnp.int32, 16)

    def group_body(g, carry):
        acc = jnp.zeros((16,), jnp.float32)
        for r in range(16):
            i = g * 16 + r
            a0 = srows[i, pl.ds(0, 16)] * trows[i, pl.ds(0, 16)]
            a1 = srows[i, pl.ds(16, 16)] * trows[i, pl.ds(16, 16)]
            a2 = srows[i, pl.ds(32, 16)] * trows[i, pl.ds(32, 16)]
            a3 = srows[i, pl.ds(48, 16)] * trows[i, pl.ds(48, 16)]
            s = jnp.sum((a0 + a1) + (a2 + a3))
            acc = jnp.where(lanes == r, s, acc)
        outv[pl.ds(g * 16, 16)] = acc * w + bb
        return carry

    lax.fori_loop(0, BPW // 16, group_body, 0)

    pltpu.sync_copy(outv, out_hbm.at[pl.ds(base, BPW)])


@jax.jit
def _sc_call(src3, tgt3, table, wb):
    f = pl.kernel(
        _sc_body,
        mesh=plsc.VectorSubcoreMesh(core_axis_name="c", subcore_axis_name="s"),
        out_type=jax.ShapeDtypeStruct((B,), jnp.float32),
        scratch_types=[
            pltpu.VMEM((NCHUNK, CH), jnp.int32),
            pltpu.VMEM((NCHUNK, CH), jnp.int32),
            pltpu.VMEM((BPW, D), jnp.float32),
            pltpu.VMEM((BPW, D), jnp.float32),
            pltpu.VMEM((BPW,), jnp.float32),
            pltpu.VMEM((16,), jnp.float32),
            pltpu.SemaphoreType.DMA,
        ],
        compiler_params=pltpu.CompilerParams(
            needs_layout_passes=False, use_tc_tiling_on_sc=False),
    )
    return f(src3, tgt3, table, wb)


def kernel(sources, targets, table, W, b):
    src3 = sources.reshape(NW, NCHUNK, CH)
    tgt3 = targets.reshape(NW, NCHUNK, CH)
    wb = jnp.zeros((16,), jnp.float32)
    wb = wb.at[0].set(W.reshape(())).at[1].set(b.reshape(()))
    out = _sc_call(src3, tgt3, table, wb)
    return out.reshape(B, 1)
